# Initial kernel scaffold; baseline (speedup 1.0000x reference)
import jax
import jax.numpy as jnp
from jax.experimental import pallas as pl


def kernel(indices, weight):
    raise NotImplementedError("write your pallas kernel here")


# speedup vs baseline: 1.1030x; 1.1030x over previous
import functools

import jax
import jax.numpy as jnp
from jax import lax
from jax.experimental import pallas as pl
from jax.experimental.pallas import tpu as pltpu
from jax.experimental.pallas import tpu_sc as plsc


_NUM_ROWS = 1000000
_DIM = 32
_B = 16384 * 50

_NC = 2
_NS = 16
_NW = _NC * _NS
_B_PER_W = _B // _NW
_CHUNK = 1600
_NCHUNKS = _B_PER_W // _CHUNK


def _gather_body(idx_hbm, table_hbm, out_hbm, idx_v, rows_v, sem):
    wid = lax.axis_index("s") * _NC + lax.axis_index("c")
    base = wid * _B_PER_W

    def chunk(g, _):
        off = base + g * _CHUNK
        pltpu.sync_copy(idx_hbm.at[pl.ds(off, _CHUNK)], idx_v)
        pltpu.async_copy(table_hbm.at[idx_v], rows_v, sem).wait()
        pltpu.sync_copy(rows_v, out_hbm.at[pl.ds(off, _CHUNK)])
        return _

    lax.fori_loop(0, _NCHUNKS, chunk, 0)


@jax.jit
def _embedding_gather(indices_flat, weight):
    mesh = plsc.VectorSubcoreMesh(core_axis_name="c", subcore_axis_name="s")
    run = functools.partial(
        pl.kernel,
        out_type=jax.ShapeDtypeStruct((_B, _DIM), jnp.float32),
        mesh=mesh,
        scratch_types=[
            pltpu.VMEM((_CHUNK,), jnp.int32),
            pltpu.VMEM((_CHUNK, _DIM), jnp.float32),
            pltpu.SemaphoreType.DMA,
        ],
        compiler_params=pltpu.CompilerParams(use_tc_tiling_on_sc=False),
    )(_gather_body)
    return run(indices_flat, weight)


def kernel(indices, weight):
    idx_flat = indices.reshape(-1).astype(jnp.int32)
    out = _embedding_gather(idx_flat, weight)
    return out.reshape(indices.shape + (weight.shape[1],))


# speedup vs baseline: 1.1124x; 1.0085x over previous
import functools

import jax
import jax.numpy as jnp
from jax import lax
from jax.experimental import pallas as pl
from jax.experimental.pallas import tpu as pltpu
from jax.experimental.pallas import tpu_sc as plsc


_NUM_ROWS = 1000000
_DIM = 32
_B = 16384 * 50

_NC = 2
_NS = 16
_NW = _NC * _NS
_B_PER_W = _B // _NW
_CHUNK = 1600
_NCHUNKS = _B_PER_W // _CHUNK


def _gather_body(idx_hbm, table_hbm, out_hbm, idx_v, rows_v,
                 gsem0, gsem1, wsem0, wsem1):
    wid = lax.axis_index("s") * _NC + lax.axis_index("c")
    base = wid * _B_PER_W
    gsems = (gsem0, gsem1)
    wsems = (wsem0, wsem1)

    gathers = [None, None]
    writebacks = [None, None]

    pltpu.sync_copy(idx_hbm.at[pl.ds(base, _CHUNK)], idx_v.at[0])
    gathers[0] = pltpu.async_copy(table_hbm.at[idx_v.at[0]], rows_v.at[0],
                                  gsems[0])

    for g in range(_NCHUNKS):
        b, nb = g % 2, (g + 1) % 2
        if g + 1 < _NCHUNKS:
            off_n = base + (g + 1) * _CHUNK
            pltpu.sync_copy(idx_hbm.at[pl.ds(off_n, _CHUNK)], idx_v.at[nb])
            if writebacks[nb] is not None:
                writebacks[nb].wait()
            gathers[nb] = pltpu.async_copy(table_hbm.at[idx_v.at[nb]],
                                           rows_v.at[nb], gsems[nb])
        gathers[b].wait()
        writebacks[b] = pltpu.async_copy(
            rows_v.at[b], out_hbm.at[pl.ds(base + g * _CHUNK, _CHUNK)],
            wsems[b])

    writebacks[0].wait()
    writebacks[1].wait()


@jax.jit
def _embedding_gather(indices_flat, weight):
    mesh = plsc.VectorSubcoreMesh(core_axis_name="c", subcore_axis_name="s")
    run = functools.partial(
        pl.kernel,
        out_type=jax.ShapeDtypeStruct((_B, _DIM), jnp.float32),
        mesh=mesh,
        scratch_types=[
            pltpu.VMEM((2, _CHUNK), jnp.int32),
            pltpu.VMEM((2, _CHUNK, _DIM), jnp.float32),
            pltpu.SemaphoreType.DMA,
            pltpu.SemaphoreType.DMA,
            pltpu.SemaphoreType.DMA,
            pltpu.SemaphoreType.DMA,
        ],
        compiler_params=pltpu.CompilerParams(use_tc_tiling_on_sc=False),
    )(_gather_body)
    return run(indices_flat, weight)


def kernel(indices, weight):
    idx_flat = indices.reshape(-1).astype(jnp.int32)
    out = _embedding_gather(idx_flat, weight)
    return out.reshape(indices.shape + (weight.shape[1],))


# speedup vs baseline: 1.5687x; 1.4102x over previous
import functools

import jax
import jax.numpy as jnp
from jax import lax
from jax.experimental import pallas as pl
from jax.experimental.pallas import tpu as pltpu
from jax.experimental.pallas import tpu_sc as plsc


_NUM_ROWS = 1000000
_DIM = 32
_NJ = 50
_NB = 16384

_NC = 2
_NS = 16
_NW = _NC * _NS
_BPW = _NB // _NW
_BT = _BPW // 128


def _transpose_unit(rows_ref, t_ref):
    iota = lax.iota(jnp.int32, 16)

    def body(m, carry):
        dt = m >> 8
        bt = (m >> 6) & 3
        di = (m >> 3) & 7
        bi0 = m & 7
        rows = bt * 128 + bi0 * 16 + iota
        cols = jnp.full((16,), dt * 8 + di, jnp.int32)
        v = plsc.load_gather(rows_ref, [rows, cols])
        t_ref[dt, bt, di, pl.ds(bi0 * 16, 16)] = v
        return carry

    lax.fori_loop(0, 1024, body, 0)


def _gather_body(idx_hbm, table_hbm, out_hbm, idx_v, rows_v0, rows_v1,
                 t_v0, t_v1, gsem0, gsem1, wsem0, wsem1):
    wid = lax.axis_index("s") * _NC + lax.axis_index("c")
    b0 = wid * _BPW
    rows_v = (rows_v0, rows_v1)
    t_v = (t_v0, t_v1)
    gsems = (gsem0, gsem1)
    wsems = (wsem0, wsem1)

    pltpu.sync_copy(idx_hbm.at[:, pl.ds(b0, _BPW)], idx_v)

    def g_start(j, b):
        pltpu.async_copy(table_hbm.at[idx_v.at[j]], rows_v[b], gsems[b])

    def g_wait(b):
        pltpu.make_async_copy(table_hbm.at[pl.ds(0, _BPW)], rows_v[b],
                              gsems[b]).wait()

    def w_start(j, b):
        pltpu.async_copy(t_v[b], out_hbm.at[j, :, pl.ds(wid * _BT, _BT)],
                         wsems[b])

    def w_wait(b):
        pltpu.make_async_copy(t_v[b], out_hbm.at[0, :, pl.ds(wid * _BT, _BT)],
                              wsems[b]).wait()

    g_start(0, 0)
    g_start(1, 1)
    g_wait(0)
    _transpose_unit(rows_v[0], t_v[0])
    w_start(0, 0)
    g_start(2, 0)
    g_wait(1)
    _transpose_unit(rows_v[1], t_v[1])
    w_start(1, 1)

    def steady(k, carry):
        j = 2 + 2 * k
        for b in (0, 1):
            jj = j + b
            nb = 1 - b
            g_start(jj + 1, nb)
            g_wait(b)
            w_wait(b)
            _transpose_unit(rows_v[b], t_v[b])
            w_start(jj, b)
        return carry

    lax.fori_loop(0, 23, steady, 0)

    g_start(49, 1)
    g_wait(0)
    w_wait(0)
    _transpose_unit(rows_v[0], t_v[0])
    w_start(48, 0)
    g_wait(1)
    w_wait(1)
    _transpose_unit(rows_v[1], t_v[1])
    w_start(49, 1)

    w_wait(0)
    w_wait(1)


@jax.jit
def _embedding_gather(idx_t, weight):
    mesh = plsc.VectorSubcoreMesh(core_axis_name="c", subcore_axis_name="s")
    run = functools.partial(
        pl.kernel,
        out_type=jax.ShapeDtypeStruct((_NJ, _DIM // 8, _NB // 128, 8, 128),
                                      jnp.float32),
        mesh=mesh,
        scratch_types=[
            pltpu.VMEM((_NJ, _BPW), jnp.int32),
            pltpu.VMEM((_BPW, _DIM), jnp.float32),
            pltpu.VMEM((_BPW, _DIM), jnp.float32),
            pltpu.VMEM((_DIM // 8, _BT, 8, 128), jnp.float32),
            pltpu.VMEM((_DIM // 8, _BT, 8, 128), jnp.float32),
            pltpu.SemaphoreType.DMA,
            pltpu.SemaphoreType.DMA,
            pltpu.SemaphoreType.DMA,
            pltpu.SemaphoreType.DMA,
        ],
        compiler_params=pltpu.CompilerParams(use_tc_tiling_on_sc=False,
                                             needs_layout_passes=False),
    )(_gather_body)
    return run(idx_t, weight)


def kernel(indices, weight):
    idx_t = indices.T.astype(jnp.int32)
    out5 = _embedding_gather(idx_t, weight)
    return out5.transpose(2, 4, 0, 1, 3).reshape(_NB, _NJ, _DIM)


# speedup vs baseline: 2.2221x; 1.4166x over previous
import functools

import jax
import jax.numpy as jnp
from jax import lax
from jax.experimental import pallas as pl
from jax.experimental.pallas import tpu as pltpu
from jax.experimental.pallas import tpu_sc as plsc


_NUM_ROWS = 1000000
_DIM = 32
_NJ = 50
_NB = 16384

_NC = 2
_NS = 16
_NW = _NC * _NS
_BPW = _NB // _NW
_BT = _BPW // 128


def _transpose_unit(rows_ref, t_ref):
    iota = lax.iota(jnp.int32, 16)

    @plsc.parallel_loop(0, 1024, step=8, unroll=2)
    def block(m):
        dt = m >> 8
        bt = (m >> 6) & 3
        di = (m >> 3) & 7
        cols = jnp.full((16,), dt * 8 + di, jnp.int32)
        rows0 = bt * 128 + iota
        off = di * 128
        for k in range(8):
            v = plsc.load_gather(rows_ref, [rows0 + k * 16, cols])
            t_ref[dt, bt, pl.ds(off + k * 16, 16)] = v


def _gather_body(idx_hbm, table_hbm, out_hbm, idx_v, rows_v0, rows_v1,
                 t_v0, t_v1, gsem0, gsem1, wsem0, wsem1):
    wid = lax.axis_index("s") * _NC + lax.axis_index("c")
    b0 = wid * _BPW
    rows_v = (rows_v0, rows_v1)
    t_v = (t_v0, t_v1)
    gsems = (gsem0, gsem1)
    wsems = (wsem0, wsem1)

    pltpu.sync_copy(idx_hbm.at[:, pl.ds(b0, _BPW)], idx_v)

    def g_start(j, b):
        pltpu.async_copy(table_hbm.at[idx_v.at[j]], rows_v[b], gsems[b])

    def g_wait(b):
        pltpu.make_async_copy(table_hbm.at[pl.ds(0, _BPW)], rows_v[b],
                              gsems[b]).wait()

    def w_start(j, b):
        pltpu.async_copy(t_v[b], out_hbm.at[j, :, pl.ds(wid * _BT, _BT)],
                         wsems[b])

    def w_wait(b):
        pltpu.make_async_copy(t_v[b], out_hbm.at[0, :, pl.ds(wid * _BT, _BT)],
                              wsems[b]).wait()

    g_start(0, 0)
    g_start(1, 1)
    g_wait(0)
    _transpose_unit(rows_v[0], t_v[0])
    w_start(0, 0)
    g_start(2, 0)
    g_wait(1)
    _transpose_unit(rows_v[1], t_v[1])
    w_start(1, 1)

    def steady(k, carry):
        j = 2 + 2 * k
        for b in (0, 1):
            jj = j + b
            nb = 1 - b
            g_start(jj + 1, nb)
            g_wait(b)
            w_wait(b)
            _transpose_unit(rows_v[b], t_v[b])
            w_start(jj, b)
        return carry

    lax.fori_loop(0, 23, steady, 0)

    g_start(49, 1)
    g_wait(0)
    w_wait(0)
    _transpose_unit(rows_v[0], t_v[0])
    w_start(48, 0)
    g_wait(1)
    w_wait(1)
    _transpose_unit(rows_v[1], t_v[1])
    w_start(49, 1)

    w_wait(0)
    w_wait(1)


@jax.jit
def _embedding_gather(idx_t, weight):
    mesh = plsc.VectorSubcoreMesh(core_axis_name="c", subcore_axis_name="s")
    run = functools.partial(
        pl.kernel,
        out_type=jax.ShapeDtypeStruct((_NJ, _DIM // 8, _NB // 128, 1024),
                                      jnp.float32),
        mesh=mesh,
        scratch_types=[
            pltpu.VMEM((_NJ, _BPW), jnp.int32),
            pltpu.VMEM((_BPW, _DIM), jnp.float32),
            pltpu.VMEM((_BPW, _DIM), jnp.float32),
            pltpu.VMEM((_DIM // 8, _BT, 1024), jnp.float32),
            pltpu.VMEM((_DIM // 8, _BT, 1024), jnp.float32),
            pltpu.SemaphoreType.DMA,
            pltpu.SemaphoreType.DMA,
            pltpu.SemaphoreType.DMA,
            pltpu.SemaphoreType.DMA,
        ],
        compiler_params=pltpu.CompilerParams(use_tc_tiling_on_sc=False,
                                             needs_layout_passes=False),
    )(_gather_body)
    return run(idx_t, weight)


def kernel(indices, weight):
    idx_t = indices.T.astype(jnp.int32)
    out4 = _embedding_gather(idx_t, weight)
    out5 = out4.reshape(_NJ, _DIM // 8, _NB // 128, 8, 128)
    return out5.transpose(2, 4, 0, 1, 3).reshape(_NB, _NJ, _DIM)


# speedup vs baseline: 3.1375x; 1.4119x over previous
import functools

import jax
import jax.numpy as jnp
from jax import lax
from jax.experimental import pallas as pl
from jax.experimental.pallas import tpu as pltpu
from jax.experimental.pallas import tpu_sc as plsc


_NUM_ROWS = 1000000
_DIM = 32
_NJ = 50
_NB = 16384

_NC = 2
_NS = 16
_NW = _NC * _NS
_BPW = _NB // _NW
_BT = _BPW // 128


def _transpose_unit(rows_ref, t_ref):
    iota = lax.iota(jnp.int32, 16)
    di_v = iota & 7
    dt_lo = iota >> 3
    dt_hi = dt_lo + 2

    for bt in range(4):
        bt_v = jnp.full((16,), bt, jnp.int32)

        @plsc.parallel_loop(0, 128, unroll=2)
        def scatter_row(bi, bt=bt, bt_v=bt_v):
            bi_v = jnp.full((16,), bi, jnp.int32)
            row = bt * 128 + bi
            v0 = rows_ref[row, pl.ds(0, 16)]
            v1 = rows_ref[row, pl.ds(16, 16)]
            plsc.store_scatter(t_ref, [bt_v, dt_lo, di_v, bi_v], v0)
            plsc.store_scatter(t_ref, [bt_v, dt_hi, di_v, bi_v], v1)


def _gather_body(idx_hbm, table_hbm, out_hbm, idx_v, rows_v0, rows_v1,
                 t_v0, t_v1, gsem0, gsem1, wsem0, wsem1):
    wid = lax.axis_index("s") * _NC + lax.axis_index("c")
    b0 = wid * _BPW
    rows_v = (rows_v0, rows_v1)
    t_v = (t_v0, t_v1)
    gsems = (gsem0, gsem1)
    wsems = (wsem0, wsem1)

    pltpu.sync_copy(idx_hbm.at[:, pl.ds(b0, _BPW)], idx_v)

    def g_start(j, b):
        pltpu.async_copy(table_hbm.at[idx_v.at[j]], rows_v[b], gsems[b])

    def g_wait(b):
        pltpu.make_async_copy(table_hbm.at[pl.ds(0, _BPW)], rows_v[b],
                              gsems[b]).wait()

    def w_start(j, b):
        for bt in range(4):
            pltpu.async_copy(t_v[b].at[bt, :, :, pl.ds(0, 128)],
                             out_hbm.at[j, :, wid * _BT + bt], wsems[b])

    def w_wait(b):
        for bt in range(4):
            pltpu.make_async_copy(t_v[b].at[bt, :, :, pl.ds(0, 128)],
                                  out_hbm.at[0, :, wid * _BT + bt],
                                  wsems[b]).wait()

    g_start(0, 0)
    g_start(1, 1)
    g_wait(0)
    _transpose_unit(rows_v[0], t_v[0])
    w_start(0, 0)
    g_start(2, 0)
    g_wait(1)
    _transpose_unit(rows_v[1], t_v[1])
    w_start(1, 1)

    def steady(k, carry):
        j = 2 + 2 * k
        for b in (0, 1):
            jj = j + b
            nb = 1 - b
            g_start(jj + 1, nb)
            g_wait(b)
            w_wait(b)
            _transpose_unit(rows_v[b], t_v[b])
            w_start(jj, b)
        return carry

    lax.fori_loop(0, 23, steady, 0)

    g_start(49, 1)
    g_wait(0)
    w_wait(0)
    _transpose_unit(rows_v[0], t_v[0])
    w_start(48, 0)
    g_wait(1)
    w_wait(1)
    _transpose_unit(rows_v[1], t_v[1])
    w_start(49, 1)

    w_wait(0)
    w_wait(1)


@jax.jit
def _embedding_gather(idx_t, weight):
    mesh = plsc.VectorSubcoreMesh(core_axis_name="c", subcore_axis_name="s")
    run = functools.partial(
        pl.kernel,
        out_type=jax.ShapeDtypeStruct((_NJ, _DIM // 8, _NB // 128, 8, 128),
                                      jnp.float32),
        mesh=mesh,
        scratch_types=[
            pltpu.VMEM((_NJ, _BPW), jnp.int32),
            pltpu.VMEM((_BPW, _DIM), jnp.float32),
            pltpu.VMEM((_BPW, _DIM), jnp.float32),
            pltpu.VMEM((_BT, _DIM // 8, 8, 129), jnp.float32),
            pltpu.VMEM((_BT, _DIM // 8, 8, 129), jnp.float32),
            pltpu.SemaphoreType.DMA,
            pltpu.SemaphoreType.DMA,
            pltpu.SemaphoreType.DMA,
            pltpu.SemaphoreType.DMA,
        ],
        compiler_params=pltpu.CompilerParams(use_tc_tiling_on_sc=False,
                                             needs_layout_passes=False),
    )(_gather_body)
    return run(idx_t, weight)


def kernel(indices, weight):
    idx_t = indices.T.astype(jnp.int32)
    out5 = _embedding_gather(idx_t, weight)
    return out5.transpose(2, 4, 0, 1, 3).reshape(_NB, _NJ, _DIM)


# speedup vs baseline: 3.1483x; 1.0035x over previous
import functools

import jax
import jax.numpy as jnp
from jax import lax
from jax.experimental import pallas as pl
from jax.experimental.pallas import tpu as pltpu
from jax.experimental.pallas import tpu_sc as plsc


_NUM_ROWS = 1000000
_DIM = 32
_NJ = 50
_NB = 16384

_NC = 2
_NS = 16
_NW = _NC * _NS
_BPW = _NB // _NW
_BT = _BPW // 128


def _transpose_unit(rows_ref, t_ref):
    iota = lax.iota(jnp.int32, 16)
    di_v = iota & 7
    dt_lo = iota >> 3
    dt_hi = dt_lo + 2

    for bt in range(4):
        bt_v = jnp.full((16,), bt, jnp.int32)

        @plsc.parallel_loop(0, 128, unroll=4)
        def scatter_row(bi, bt=bt, bt_v=bt_v):
            bi_v = jnp.full((16,), bi, jnp.int32)
            row = bt * 128 + bi
            v0 = rows_ref[row, pl.ds(0, 16)]
            v1 = rows_ref[row, pl.ds(16, 16)]
            plsc.store_scatter(t_ref, [bt_v, dt_lo, di_v, bi_v], v0)
            plsc.store_scatter(t_ref, [bt_v, dt_hi, di_v, bi_v], v1)


def _gather_body(idx_hbm, table_hbm, out_hbm, idx_v, rows_v0, rows_v1,
                 t_v0, t_v1, gsem0, gsem1, wsem0, wsem1):
    wid = lax.axis_index("s") * _NC + lax.axis_index("c")
    b0 = wid * _BPW
    rows_v = (rows_v0, rows_v1)
    t_v = (t_v0, t_v1)
    gsems = (gsem0, gsem1)
    wsems = (wsem0, wsem1)

    pltpu.sync_copy(idx_hbm.at[:, pl.ds(b0, _BPW)], idx_v)

    def g_start(j, b):
        pltpu.async_copy(table_hbm.at[idx_v.at[j]], rows_v[b], gsems[b])

    def g_wait(b):
        pltpu.make_async_copy(table_hbm.at[pl.ds(0, _BPW)], rows_v[b],
                              gsems[b]).wait()

    def w_start(j, b):
        for bt in range(4):
            pltpu.async_copy(t_v[b].at[bt, :, :, pl.ds(0, 128)],
                             out_hbm.at[j, :, wid * _BT + bt], wsems[b])

    def w_wait(b):
        for bt in range(4):
            pltpu.make_async_copy(t_v[b].at[bt, :, :, pl.ds(0, 128)],
                                  out_hbm.at[0, :, wid * _BT + bt],
                                  wsems[b]).wait()

    g_start(0, 0)
    g_start(1, 1)
    g_wait(0)
    _transpose_unit(rows_v[0], t_v[0])
    w_start(0, 0)
    g_start(2, 0)
    g_wait(1)
    _transpose_unit(rows_v[1], t_v[1])
    w_start(1, 1)

    def steady(k, carry):
        j = 2 + 2 * k
        for b in (0, 1):
            jj = j + b
            nb = 1 - b
            g_start(jj + 1, nb)
            g_wait(b)
            w_wait(b)
            _transpose_unit(rows_v[b], t_v[b])
            w_start(jj, b)
        return carry

    lax.fori_loop(0, 23, steady, 0)

    g_start(49, 1)
    g_wait(0)
    w_wait(0)
    _transpose_unit(rows_v[0], t_v[0])
    w_start(48, 0)
    g_wait(1)
    w_wait(1)
    _transpose_unit(rows_v[1], t_v[1])
    w_start(49, 1)

    w_wait(0)
    w_wait(1)


@jax.jit
def _embedding_gather(idx_t, weight):
    mesh = plsc.VectorSubcoreMesh(core_axis_name="c", subcore_axis_name="s")
    run = functools.partial(
        pl.kernel,
        out_type=jax.ShapeDtypeStruct((_NJ, _DIM // 8, _NB // 128, 8, 128),
                                      jnp.float32),
        mesh=mesh,
        scratch_types=[
            pltpu.VMEM((_NJ, _BPW), jnp.int32),
            pltpu.VMEM((_BPW, _DIM), jnp.float32),
            pltpu.VMEM((_BPW, _DIM), jnp.float32),
            pltpu.VMEM((_BT, _DIM // 8, 8, 129), jnp.float32),
            pltpu.VMEM((_BT, _DIM // 8, 8, 129), jnp.float32),
            pltpu.SemaphoreType.DMA,
            pltpu.SemaphoreType.DMA,
            pltpu.SemaphoreType.DMA,
            pltpu.SemaphoreType.DMA,
        ],
        compiler_params=pltpu.CompilerParams(use_tc_tiling_on_sc=False,
                                             needs_layout_passes=False),
    )(_gather_body)
    return run(idx_t, weight)


def kernel(indices, weight):
    idx_t = indices.T.astype(jnp.int32)
    out5 = _embedding_gather(idx_t, weight)
    return out5.transpose(2, 4, 0, 1, 3).reshape(_NB, _NJ, _DIM)
